# Initial kernel scaffold; baseline (speedup 1.0000x reference)
import jax
import jax.numpy as jnp
from jax.experimental import pallas as pl


def kernel(hidden_states, task_id, mode_id, W_gate, task_bias, mode_bias, A, Bw):
    raise NotImplementedError("write your pallas kernel here")


# speedup vs baseline: 8.0614x; 8.0614x over previous
import jax
import jax.numpy as jnp
from jax.experimental import pallas as pl
from jax.experimental.pallas import tpu as pltpu

_ALPHA = 16.0


def _moe_lora_body(x_ref, bias_ref, wg_ref, acat_ref, bcat_ref, out_ref,
                   pooled_ref, wrep_ref, h_ref, *, ts, nt, s_total, e, r):
    phase = pl.program_id(1)
    s = pl.program_id(2)

    @pl.when(phase == 0)
    def _phase0():
        x = x_ref[0]
        part = jnp.sum(x, axis=0, keepdims=True)

        @pl.when(s == 0)
        def _():
            pooled_ref[...] = part

        @pl.when(s != 0)
        def _():
            pooled_ref[...] = pooled_ref[...] + part

        h_ref[pl.ds(s * ts, ts), :] = jax.lax.dot_general(
            x, acat_ref[...], (((1,), (1,)), ((), ())),
            preferred_element_type=jnp.float32)

    @pl.when((phase == 1) & (s == 0))
    def _router():
        pooled = pooled_ref[...] * (1.0 / s_total)
        logits_col = jnp.sum(wg_ref[...] * pooled, axis=1, keepdims=True)
        logits_col = logits_col + bias_ref[0]
        iota = jax.lax.broadcasted_iota(jnp.int32, (e, 1), 0)
        v0 = jnp.max(logits_col, keepdims=True)
        i0 = jnp.min(jnp.where(logits_col == v0, iota, e), keepdims=True)
        masked = jnp.where(iota == i0, -jnp.inf, logits_col)
        v1 = jnp.max(masked, keepdims=True)
        i1 = jnp.min(jnp.where(masked == v1, iota, e), keepdims=True)
        t = jnp.exp(v1 - v0)
        w0 = 1.0 / (1.0 + t)
        w1 = t / (1.0 + t)
        scaling = _ALPHA / r
        eidx = jax.lax.broadcasted_iota(jnp.int32, (1, e * r), 1) // r
        wrep = (jnp.where(eidx == i0, w0 * scaling, 0.0)
                + jnp.where(eidx == i1, w1 * scaling, 0.0))
        wrep_ref[...] = wrep

    @pl.when(phase == 1)
    def _phase1():
        h = h_ref[pl.ds(s * ts, ts), :]
        hs = h * wrep_ref[...]
        out_ref[0] = jax.lax.dot_general(
            hs, bcat_ref[...], (((1,), (0,)), ((), ())),
            preferred_element_type=jnp.float32)


def kernel(hidden_states, task_id, mode_id, W_gate, task_bias, mode_bias, A, Bw):
    b, s_total, d = hidden_states.shape
    e, r, _ = A.shape
    ts = 512
    nt = s_total // ts

    a_cat = A.reshape(e * r, d)
    b_cat = Bw.transpose(0, 2, 1).reshape(e * r, d)
    bias = (jnp.take(task_bias, task_id, axis=0)
            + jnp.take(mode_bias, mode_id, axis=0))
    bias_col = bias.reshape(b, e, 1)

    import functools
    body = functools.partial(_moe_lora_body, ts=ts, nt=nt,
                             s_total=s_total, e=e, r=r)

    return pl.pallas_call(
        body,
        grid=(b, 2, nt),
        in_specs=[
            pl.BlockSpec((1, ts, d),
                         lambda bi, p, si: (bi, jnp.where(p == 0, si, nt - 1), 0)),
            pl.BlockSpec((1, e, 1), lambda bi, p, si: (bi, 0, 0)),
            pl.BlockSpec((e, d), lambda bi, p, si: (0, 0)),
            pl.BlockSpec((e * r, d), lambda bi, p, si: (0, 0)),
            pl.BlockSpec((e * r, d), lambda bi, p, si: (0, 0)),
        ],
        out_specs=pl.BlockSpec((1, ts, d),
                               lambda bi, p, si: (bi, jnp.where(p == 1, si, 0), 0)),
        out_shape=jax.ShapeDtypeStruct((b, s_total, d), jnp.float32),
        scratch_shapes=[
            pltpu.VMEM((1, d), jnp.float32),
            pltpu.VMEM((1, e * r), jnp.float32),
            pltpu.VMEM((s_total, e * r), jnp.float32),
        ],
    )(hidden_states, bias_col, W_gate, a_cat, b_cat)
